# baseline (speedup 1.0000x reference)
import jax
import jax.numpy as jnp
from jax.experimental import pallas as pl
from jax.experimental.pallas import tpu as pltpu

_T = 512
_K_NEIG = 10


_BB = 8


def _diag_mask_kernel(seq_ref, out_ref):
    g = pl.program_id(0)
    row = jax.lax.broadcasted_iota(jnp.int32, (_T, _T), 0)
    col = jax.lax.broadcasted_iota(jnp.int32, (_T, _T), 1)
    diag = row == col
    for i in range(_BB):
        thresh = jnp.maximum(seq_ref[g * _BB + i], _T - _K_NEIG + 1)
        mask = diag & (row < thresh)
        out_ref[i] = mask.astype(jnp.float32)


def kernel(input, seq_len, weight0, weight1):
    B = input.shape[0]
    grid_spec = pltpu.PrefetchScalarGridSpec(
        num_scalar_prefetch=1,
        grid=(B // _BB,),
        out_specs=pl.BlockSpec((_BB, _T, _T), lambda g, s: (g, 0, 0)),
    )
    return pl.pallas_call(
        _diag_mask_kernel,
        grid_spec=grid_spec,
        out_shape=jax.ShapeDtypeStruct((B, _T, _T), jnp.float32),
    )(seq_len.astype(jnp.int32))


# speedup vs baseline: 1.0850x; 1.0850x over previous
import jax
import jax.numpy as jnp
from jax.experimental import pallas as pl
from jax.experimental.pallas import tpu as pltpu

_T = 512
_K_NEIG = 10


_BB = 4


def _diag_mask_kernel(seq_ref, out_ref):
    g = pl.program_id(0)
    row = jax.lax.broadcasted_iota(jnp.int32, (_T, _T), 0)
    col = jax.lax.broadcasted_iota(jnp.int32, (_T, _T), 1)
    diag = row == col
    for i in range(_BB):
        thresh = jnp.maximum(seq_ref[g * _BB + i], _T - _K_NEIG + 1)
        mask = diag & (row < thresh)
        out_ref[i] = mask.astype(jnp.float32)


def kernel(input, seq_len, weight0, weight1):
    B = input.shape[0]
    grid_spec = pltpu.PrefetchScalarGridSpec(
        num_scalar_prefetch=1,
        grid=(B // _BB,),
        out_specs=pl.BlockSpec((_BB, _T, _T), lambda g, s: (g, 0, 0)),
    )
    return pl.pallas_call(
        _diag_mask_kernel,
        grid_spec=grid_spec,
        out_shape=jax.ShapeDtypeStruct((B, _T, _T), jnp.float32),
    )(seq_len.astype(jnp.int32))
